# baseline (speedup 1.0000x reference)
import jax
import jax.numpy as jnp
from jax import lax
from jax.experimental import pallas as pl
from jax.experimental.pallas import tpu as pltpu

_BN = 512
_MB = 4
_WSLOTS = 3
_OSLOTS = 2


def _dot_nt(a, b):
    return lax.dot_general(
        a, b,
        dimension_numbers=(((1,), (1,)), ((), ())),
        preferred_element_type=jnp.float32,
        precision=lax.Precision.DEFAULT,
    )


def _make_body(batch, in_f, out_f):
    nb = out_f // _BN
    bm = batch // _MB

    def body(x_hbm, w_hbm, b_hbm, o_hbm, xs, wbuf, obuf, bbuf,
             sem_x, sem_w, sem_o, sem_b):
        cp_b = pltpu.make_async_copy(b_hbm, bbuf, sem_b)
        cp_b.start()

        x_copies = []
        for m in range(_MB):
            x_copies.append(pltpu.make_async_copy(
                x_hbm.at[pl.ds(m * bm, bm), :],
                xs.at[pl.ds(m * bm, bm), :],
                sem_x.at[m]))

        w_copies = [None] * nb

        def start_w(n):
            c = pltpu.make_async_copy(
                w_hbm.at[pl.ds(n * _BN, _BN), :],
                wbuf.at[n % _WSLOTS],
                sem_w.at[n % _WSLOTS])
            c.start()
            w_copies[n] = c

        x_copies[0].start()
        start_w(0)
        for m in range(1, _MB):
            x_copies[m].start()
        for n in range(1, _WSLOTS):
            start_w(n)
        cp_b.wait()

        o_copies = [None] * nb
        for n in range(nb):
            wslot = n % _WSLOTS
            oslot = n % _OSLOTS
            w_copies[n].wait()
            if n >= _OSLOTS:
                o_copies[n - _OSLOTS].wait()
            bslice = bbuf[:, pl.ds(n * _BN, _BN)]
            if n == 0:
                for m in range(_MB):
                    x_copies[m].wait()
                    acc = _dot_nt(xs[pl.ds(m * bm, bm), :], wbuf[wslot])
                    obuf[oslot, pl.ds(m * bm, bm), :] = acc + bslice
            else:
                obuf[oslot] = _dot_nt(xs[...], wbuf[wslot]) + bslice
            if n + _WSLOTS < nb:
                start_w(n + _WSLOTS)
            oc = pltpu.make_async_copy(
                obuf.at[oslot],
                o_hbm.at[:, pl.ds(n * _BN, _BN)],
                sem_o.at[oslot])
            oc.start()
            o_copies[n] = oc
        for n in range(nb - _OSLOTS, nb):
            o_copies[n].wait()

    return body


def kernel(x, weight, bias):
    batch, in_f = x.shape
    out_f = weight.shape[0]
    brow = bias.reshape(1, out_f)
    return pl.pallas_call(
        _make_body(batch, in_f, out_f),
        in_specs=[pl.BlockSpec(memory_space=pl.ANY)] * 3,
        out_specs=pl.BlockSpec(memory_space=pl.ANY),
        out_shape=jax.ShapeDtypeStruct((batch, out_f), jnp.float32),
        scratch_shapes=[
            pltpu.VMEM((batch, in_f), jnp.float32),
            pltpu.VMEM((_WSLOTS, _BN, in_f), jnp.float32),
            pltpu.VMEM((_OSLOTS, batch, _BN), jnp.float32),
            pltpu.VMEM((1, out_f), jnp.float32),
            pltpu.SemaphoreType.DMA((_MB,)),
            pltpu.SemaphoreType.DMA((_WSLOTS,)),
            pltpu.SemaphoreType.DMA((_OSLOTS,)),
            pltpu.SemaphoreType.DMA,
        ],
    )(x, weight, brow)


# speedup vs baseline: 1.1193x; 1.1193x over previous
import jax
import jax.numpy as jnp
from jax import lax
from jax.experimental import pallas as pl
from jax.experimental.pallas import tpu as pltpu

_BN = 512


def _linear_kernel(x_ref, w_ref, b_ref, o_ref):
    acc = lax.dot_general(
        x_ref[...], w_ref[...],
        dimension_numbers=(((1,), (1,)), ((), ())),
        preferred_element_type=jnp.float32,
        precision=lax.Precision.DEFAULT,
    )
    o_ref[...] = acc + b_ref[...]


def kernel(x, weight, bias):
    batch, in_f = x.shape
    out_f = weight.shape[0]
    brow = bias.reshape(1, out_f)
    return pl.pallas_call(
        _linear_kernel,
        grid=(out_f // _BN,),
        in_specs=[
            pl.BlockSpec((batch, in_f), lambda n: (0, 0),
                         pipeline_mode=pl.Buffered(buffer_count=1)),
            pl.BlockSpec((_BN, in_f), lambda n: (n, 0),
                         pipeline_mode=pl.Buffered(buffer_count=2)),
            pl.BlockSpec((1, _BN), lambda n: (0, n)),
        ],
        out_specs=pl.BlockSpec((batch, _BN), lambda n: (0, n)),
        out_shape=jax.ShapeDtypeStruct((batch, out_f), jnp.float32),
        compiler_params=pltpu.CompilerParams(
            dimension_semantics=("arbitrary",),
        ),
    )(x, weight, brow)
